# baseline (speedup 1.0000x reference)
import jax
import jax.numpy as jnp
from jax.experimental import pallas as pl


def _body(x_ref, ww_ref, wc_ref, o_ref):
    o_ref[...] = x_ref[...] + ww_ref[...][None] + wc_ref[...][:, None, :]


def kernel(input_embeddings, W_word, W_char):
    C, W, B, D = input_embeddings.shape
    BD = B * D
    x = input_embeddings.reshape(C, W, BD)
    ww = jnp.tile(W_word, (1, B))
    wc = jnp.tile(W_char, (1, B))

    BC = 8
    out = pl.pallas_call(
        _body,
        grid=(C // BC,),
        in_specs=[
            pl.BlockSpec((BC, W, BD), lambda i: (i, 0, 0)),
            pl.BlockSpec((W, BD), lambda i: (0, 0)),
            pl.BlockSpec((BC, BD), lambda i: (i, 0)),
        ],
        out_specs=pl.BlockSpec((BC, W, BD), lambda i: (i, 0, 0)),
        out_shape=jax.ShapeDtypeStruct((C, W, BD), jnp.float32),
    )(x, ww, wc)
    return out.reshape(C, W, B, D)
